# Initial kernel scaffold; baseline (speedup 1.0000x reference)
import jax
import jax.numpy as jnp
from jax.experimental import pallas as pl


def kernel(x):
    raise NotImplementedError("write your pallas kernel here")


# speedup vs baseline: 16.7875x; 16.7875x over previous
import functools

import jax
import jax.numpy as jnp
from jax import lax
from jax.experimental import pallas as pl

_K = 16
_BR = 256


def _knn_body(q_ref, k_ref, nn_ref, ct_ref, *, n, k):
    b = pl.program_id(0)
    q = q_ref[0]
    kt = k_ref[0]

    c_dim = q.shape[1]

    qs = q[:, 0:1] * q[:, 0:1]
    for c in range(1, c_dim):
        qs = qs + q[:, c:c + 1] * q[:, c:c + 1]
    qn = q / jnp.maximum(jnp.sqrt(qs), 1e-12)

    ks = kt[0:1] * kt[0:1]
    for c in range(1, c_dim):
        ks = ks + kt[c:c + 1] * kt[c:c + 1]
    knt = kt / jnp.maximum(jnp.sqrt(ks), 1e-12)

    dots = lax.dot_general(qn, knt, (((1,), (0,)), ((), ())),
                           preferred_element_type=jnp.float32)

    sq_q = qn[:, 0:1] * qn[:, 0:1]
    for c in range(1, c_dim):
        sq_q = sq_q + qn[:, c:c + 1] * qn[:, c:c + 1]
    sq_k = knt[0:1] * knt[0:1]
    for c in range(1, c_dim):
        sq_k = sq_k + knt[c:c + 1] * knt[c:c + 1]

    dist = (sq_q + (-2.0 * dots)) + sq_k

    iota = lax.broadcasted_iota(jnp.int32, dist.shape, 1)
    work = dist
    cols = []
    for t in range(k + 1):
        m = jnp.min(work, axis=1, keepdims=True)
        eq = work == m
        idx = jnp.min(jnp.where(eq, iota, n), axis=1, keepdims=True)
        if t > 0:
            cols.append(idx)
        work = jnp.where(iota == idx, jnp.inf, work)

    off = b * n
    nn_ref[0] = jnp.concatenate(cols, axis=1) + off
    row0 = pl.program_id(1) * _BR + off
    ct_ref[0] = lax.broadcasted_iota(jnp.int32, (_BR, k), 0) + row0


def kernel(x):
    B, C, H, W = x.shape
    n = H * W
    xc = x.reshape(B, C, n)
    xt = jnp.transpose(xc, (0, 2, 1))

    grid = (B, n // _BR)
    nn, ct = pl.pallas_call(
        functools.partial(_knn_body, n=n, k=_K),
        grid=grid,
        in_specs=[
            pl.BlockSpec((1, _BR, C), lambda b, r: (b, r, 0)),
            pl.BlockSpec((1, C, n), lambda b, r: (b, 0, 0)),
        ],
        out_specs=[
            pl.BlockSpec((1, _BR, _K), lambda b, r: (b, r, 0)),
            pl.BlockSpec((1, _BR, _K), lambda b, r: (b, r, 0)),
        ],
        out_shape=[
            jax.ShapeDtypeStruct((B, n, _K), jnp.int32),
            jax.ShapeDtypeStruct((B, n, _K), jnp.int32),
        ],
    )(xt, xc)
    return jnp.stack((nn.reshape(-1), ct.reshape(-1)), axis=0)


# speedup vs baseline: 20.0170x; 1.1924x over previous
import functools

import jax
import jax.numpy as jnp
from jax import lax
from jax.experimental import pallas as pl
from jax.experimental.pallas import tpu as pltpu
from jax.experimental.pallas import tpu_sc as plsc

_K = 16
_BR = 256
_L = 16
_GRP = 256


def _dist_body(q_ref, k_ref, d_ref, ct_ref, *, n, k):
    b = pl.program_id(0)
    q = q_ref[0]
    kt = k_ref[0]
    c_dim = q.shape[1]

    qs = q[:, 0:1] * q[:, 0:1]
    for c in range(1, c_dim):
        qs = qs + q[:, c:c + 1] * q[:, c:c + 1]
    qn = q / jnp.maximum(jnp.sqrt(qs), 1e-12)

    ks = kt[0:1] * kt[0:1]
    for c in range(1, c_dim):
        ks = ks + kt[c:c + 1] * kt[c:c + 1]
    knt = kt / jnp.maximum(jnp.sqrt(ks), 1e-12)

    dots = lax.dot_general(qn, knt, (((1,), (0,)), ((), ())),
                           preferred_element_type=jnp.float32)

    sq_q = qn[:, 0:1] * qn[:, 0:1]
    for c in range(1, c_dim):
        sq_q = sq_q + qn[:, c:c + 1] * qn[:, c:c + 1]
    sq_k = knt[0:1] * knt[0:1]
    for c in range(1, c_dim):
        sq_k = sq_k + knt[c:c + 1] * knt[c:c + 1]

    d_ref[0] = (sq_q + (-2.0 * dots)) + sq_k

    row0 = b * n + pl.program_id(1) * _BR
    ct_ref[0] = lax.broadcasted_iota(jnp.int32, (_BR, k), 0) + row0


def _topk_sc_body(dist_hbm, nn_hbm, buf0, buf1, acc, outv,
                  sem0, sem1, *, n, nrows, k):
    nc = 2
    wid = lax.axis_index("s") * nc + lax.axis_index("c")
    rpw = nrows // 32
    base = wid * rpw
    na = n // _GRP
    lanes = jnp.arange(_L, dtype=jnp.int32)
    last = base + rpw - 1

    def fmin_s(v):
        return -(plsc.cummax(-v)[_L - 1])

    imin_s = fmin_s

    def start(buf, sem, row):
        pltpu.make_async_copy(dist_hbm.at[row], buf, sem).start()

    def wait(buf, sem):
        pltpu.make_async_copy(dist_hbm.at[base], buf, sem).wait()

    def process(buf, row):
        def fold(a, carry):
            v = buf[pl.ds(a * _GRP, _L)]
            for t in range(1, _GRP // _L):
                v = jnp.minimum(v, buf[pl.ds(a * _GRP + t * _L, _L)])
            acc[pl.ds(a * _L, _L)] = v
            return carry
        lax.fori_loop(0, na, fold, 0)

        idxs = jnp.zeros((_L,), jnp.int32)
        for t in range(k + 1):
            def fold2(a, carry):
                cur, jsel = carry
                v = acc[pl.ds(a * _L, _L)]
                lt = v < cur
                return jnp.where(lt, v, cur), jnp.where(lt, a, jsel)
            cur, jsel = lax.fori_loop(
                1, na, fold2, (acc[pl.ds(0, _L)], jnp.zeros((_L,), jnp.int32)))
            m = fmin_s(cur)
            meq = cur == m
            jstar = imin_s(jnp.where(meq, jsel, na))
            lstar = imin_s(jnp.where(meq & (jsel == jstar), lanes, _L))
            g = plsc.load_gather(buf, [lanes * _L + (jstar * _GRP + lstar)])
            tstar = imin_s(jnp.where(g == m, lanes, _L))
            vbase = jstar * _GRP + tstar * _L
            col = vbase + lstar
            vv = buf[pl.ds(vbase, _L)]
            buf[pl.ds(vbase, _L)] = jnp.where(lanes == lstar, jnp.inf, vv)
            w = buf[pl.ds(jstar * _GRP, _L)]
            for tt in range(1, _GRP // _L):
                w = jnp.minimum(w, buf[pl.ds(jstar * _GRP + tt * _L, _L)])
            acc[pl.ds(jstar * _L, _L)] = w
            if t > 0:
                idxs = jnp.where(lanes == (t - 1), col, idxs)

        off = jnp.where(row >= n, n, 0)
        outv[...] = idxs + off
        pltpu.sync_copy(outv, nn_hbm.at[row])

    start(buf0, sem0, base)
    start(buf1, sem1, base + 1)

    def outer(gi, carry):
        r0 = base + 2 * gi
        wait(buf0, sem0)
        process(buf0, r0)
        start(buf0, sem0, jnp.minimum(r0 + 2, last))
        wait(buf1, sem1)
        process(buf1, r0 + 1)
        start(buf1, sem1, jnp.minimum(r0 + 3, last))
        return carry
    lax.fori_loop(0, rpw // 2, outer, 0)
    wait(buf0, sem0)
    wait(buf1, sem1)


def kernel(x):
    B, C, H, W = x.shape
    n = H * W
    rows = B * n
    xc = x.reshape(B, C, n)
    xt = jnp.transpose(xc, (0, 2, 1))

    dist, ct = pl.pallas_call(
        functools.partial(_dist_body, n=n, k=_K),
        grid=(B, n // _BR),
        in_specs=[
            pl.BlockSpec((1, _BR, C), lambda b, r: (b, r, 0)),
            pl.BlockSpec((1, C, n), lambda b, r: (b, 0, 0)),
        ],
        out_specs=[
            pl.BlockSpec((1, _BR, n), lambda b, r: (b, r, 0)),
            pl.BlockSpec((1, _BR, _K), lambda b, r: (b, r, 0)),
        ],
        out_shape=[
            jax.ShapeDtypeStruct((B, n, n), jnp.float32),
            jax.ShapeDtypeStruct((B, n, _K), jnp.int32),
        ],
    )(xt, xc)

    mesh = plsc.VectorSubcoreMesh(core_axis_name="c", subcore_axis_name="s")
    nn = pl.kernel(
        functools.partial(_topk_sc_body, n=n, nrows=rows, k=_K),
        out_type=jax.ShapeDtypeStruct((rows, _K), jnp.int32),
        mesh=mesh,
        compiler_params=pltpu.CompilerParams(needs_layout_passes=False),
        scratch_types=[
            pltpu.VMEM((n,), jnp.float32),
            pltpu.VMEM((n,), jnp.float32),
            pltpu.VMEM((n // _L,), jnp.float32),
            pltpu.VMEM((_L,), jnp.int32),
            pltpu.SemaphoreType.DMA,
            pltpu.SemaphoreType.DMA,
        ],
    )(dist.reshape(rows, n))
    return jnp.stack((nn.reshape(-1), ct.reshape(-1)), axis=0)


# speedup vs baseline: 24.4564x; 1.2218x over previous
import functools

import jax
import jax.numpy as jnp
from jax import lax
from jax.experimental import pallas as pl
from jax.experimental.pallas import tpu as pltpu
from jax.experimental.pallas import tpu_sc as plsc

_K = 16
_BR = 256
_L = 16
_T = 8
_NU = 1152


def _dist_body(q_ref, k_ref, d_ref, a_ref, ct_ref, *, n, k):
    b = pl.program_id(0)
    q = q_ref[0]
    kt = k_ref[0]
    c_dim = q.shape[1]

    qs = q[:, 0:1] * q[:, 0:1]
    for c in range(1, c_dim):
        qs = qs + q[:, c:c + 1] * q[:, c:c + 1]
    qn = q / jnp.maximum(jnp.sqrt(qs), 1e-12)

    ks = kt[0:1] * kt[0:1]
    for c in range(1, c_dim):
        ks = ks + kt[c:c + 1] * kt[c:c + 1]
    knt = kt / jnp.maximum(jnp.sqrt(ks), 1e-12)

    dots = lax.dot_general(qn, knt, (((1,), (0,)), ((), ())),
                           preferred_element_type=jnp.float32)

    sq_q = qn[:, 0:1] * qn[:, 0:1]
    for c in range(1, c_dim):
        sq_q = sq_q + qn[:, c:c + 1] * qn[:, c:c + 1]
    sq_k = knt[0:1] * knt[0:1]
    for c in range(1, c_dim):
        sq_k = sq_k + knt[c:c + 1] * knt[c:c + 1]

    dist = (sq_q + (-2.0 * dots)) + sq_k
    d_ref[0] = dist

    acc = dist[:, 0:_NU]
    for t in range(1, _T):
        acc = jnp.minimum(acc, dist[:, t * _NU:(t + 1) * _NU])
    a_ref[0] = acc

    row0 = b * n + pl.program_id(1) * _BR
    ct_ref[0] = lax.broadcasted_iota(jnp.int32, (_BR, k), 0) + row0


def _topk_sc_body(dist_hbm, accm_hbm, nn_hbm, buf, accb, outv, semd, sema,
                  *, n, nrows, k):
    nc = 2
    wid = lax.axis_index("s") * nc + lax.axis_index("c")
    rpw = nrows // 32
    base = wid * rpw
    na = _NU // _L
    lanes = jnp.arange(_L, dtype=jnp.int32)
    tcap = jnp.minimum(lanes, _T - 1)
    last = base + rpw - 1

    def fmin_s(v):
        return -(plsc.cummax(-v)[_L - 1])

    imin_s = fmin_s

    def start(half, row):
        pltpu.make_async_copy(dist_hbm.at[row], buf.at[pl.ds(half * n, n)],
                              semd).start()
        pltpu.make_async_copy(accm_hbm.at[row],
                              accb.at[pl.ds(half * _NU, _NU)], sema).start()

    def wait():
        pltpu.make_async_copy(dist_hbm.at[base], buf.at[pl.ds(0, n)],
                              semd).wait()
        pltpu.make_async_copy(accm_hbm.at[base], accb.at[pl.ds(0, _NU)],
                              sema).wait()

    def process(bb, ab, row):
        idxs = jnp.zeros((_L,), jnp.int32)
        for t in range(k + 1):
            def fold(a8, carry):
                cur, jsel = carry
                for d in range(8):
                    a = a8 * 8 + d
                    v = accb[pl.ds(ab + a * _L, _L)]
                    lt = v < cur
                    cur = jnp.where(lt, v, cur)
                    jsel = jnp.where(lt, a, jsel)
                return cur, jsel
            big = jnp.full((_L,), jnp.inf, jnp.float32)
            cur, jsel = lax.fori_loop(0, na // 8, fold,
                                      (big, jnp.zeros((_L,), jnp.int32)))
            m = fmin_s(cur)
            meq = cur == m
            jstar = imin_s(jnp.where(meq, jsel, na))
            lstar = imin_s(jnp.where(meq & (jsel == jstar), lanes, _L))
            u = jstar * _L + lstar
            g = plsc.load_gather(buf, [bb + tcap * _NU + u])
            tstar = imin_s(jnp.where(g == m, lanes, _L))
            col = tstar * _NU + u
            vbase = col - lstar
            vv = buf[pl.ds(bb + vbase, _L)]
            buf[pl.ds(bb + vbase, _L)] = jnp.where(lanes == lstar, jnp.inf, vv)
            w = buf[pl.ds(bb + jstar * _L, _L)]
            for tt in range(1, _T):
                w = jnp.minimum(w, buf[pl.ds(bb + tt * _NU + jstar * _L, _L)])
            accb[pl.ds(ab + jstar * _L, _L)] = w
            if t > 0:
                idxs = jnp.where(lanes == (t - 1), col, idxs)

        off = jnp.where(row >= n, n, 0)
        outv[...] = idxs + off
        pltpu.sync_copy(outv, nn_hbm.at[row])

    start(0, base)

    def outer(i, carry):
        p = jnp.bitwise_and(i, 1)
        row = base + i
        wait()
        start(1 - p, jnp.minimum(row + 1, last))
        process(p * n, p * _NU, row)
        return carry
    lax.fori_loop(0, rpw, outer, 0)
    wait()


def kernel(x):
    B, C, H, W = x.shape
    n = H * W
    rows = B * n
    xc = x.reshape(B, C, n)
    xt = jnp.transpose(xc, (0, 2, 1))

    dist, accm, ct = pl.pallas_call(
        functools.partial(_dist_body, n=n, k=_K),
        grid=(B, n // _BR),
        in_specs=[
            pl.BlockSpec((1, _BR, C), lambda b, r: (b, r, 0)),
            pl.BlockSpec((1, C, n), lambda b, r: (b, 0, 0)),
        ],
        out_specs=[
            pl.BlockSpec((1, _BR, n), lambda b, r: (b, r, 0)),
            pl.BlockSpec((1, _BR, _NU), lambda b, r: (b, r, 0)),
            pl.BlockSpec((1, _BR, _K), lambda b, r: (b, r, 0)),
        ],
        out_shape=[
            jax.ShapeDtypeStruct((B, n, n), jnp.float32),
            jax.ShapeDtypeStruct((B, n, _NU), jnp.float32),
            jax.ShapeDtypeStruct((B, n, _K), jnp.int32),
        ],
    )(xt, xc)

    mesh = plsc.VectorSubcoreMesh(core_axis_name="c", subcore_axis_name="s")
    nn = pl.kernel(
        functools.partial(_topk_sc_body, n=n, nrows=rows, k=_K),
        out_type=jax.ShapeDtypeStruct((rows, _K), jnp.int32),
        mesh=mesh,
        compiler_params=pltpu.CompilerParams(needs_layout_passes=False),
        scratch_types=[
            pltpu.VMEM((2 * n,), jnp.float32),
            pltpu.VMEM((2 * _NU,), jnp.float32),
            pltpu.VMEM((_L,), jnp.int32),
            pltpu.SemaphoreType.DMA,
            pltpu.SemaphoreType.DMA,
        ],
    )(dist.reshape(rows, n), accm.reshape(rows, _NU))
    return jnp.stack((nn.reshape(-1), ct.reshape(-1)), axis=0)


# speedup vs baseline: 28.6293x; 1.1706x over previous
import functools

import jax
import jax.numpy as jnp
from jax import lax
from jax.experimental import pallas as pl
from jax.experimental.pallas import tpu as pltpu
from jax.experimental.pallas import tpu_sc as plsc

_K = 16
_BR = 256
_L = 16
_T = 8
_NU = 1152
_T2 = 9
_NU2 = 128


def _dist_body(q_ref, k_ref, d_ref, a_ref, a2_ref, ct_ref, *, n, k):
    b = pl.program_id(0)
    q = q_ref[0]
    kt = k_ref[0]
    c_dim = q.shape[1]

    qs = q[:, 0:1] * q[:, 0:1]
    for c in range(1, c_dim):
        qs = qs + q[:, c:c + 1] * q[:, c:c + 1]
    qn = q / jnp.maximum(jnp.sqrt(qs), 1e-12)

    ks = kt[0:1] * kt[0:1]
    for c in range(1, c_dim):
        ks = ks + kt[c:c + 1] * kt[c:c + 1]
    knt = kt / jnp.maximum(jnp.sqrt(ks), 1e-12)

    dots = lax.dot_general(qn, knt, (((1,), (0,)), ((), ())),
                           preferred_element_type=jnp.float32)

    sq_q = qn[:, 0:1] * qn[:, 0:1]
    for c in range(1, c_dim):
        sq_q = sq_q + qn[:, c:c + 1] * qn[:, c:c + 1]
    sq_k = knt[0:1] * knt[0:1]
    for c in range(1, c_dim):
        sq_k = sq_k + knt[c:c + 1] * knt[c:c + 1]

    dist = (sq_q + (-2.0 * dots)) + sq_k
    d_ref[0] = dist

    acc = dist[:, 0:_NU]
    for t in range(1, _T):
        acc = jnp.minimum(acc, dist[:, t * _NU:(t + 1) * _NU])
    a_ref[0] = acc

    acc2 = acc[:, 0:_NU2]
    for t in range(1, _T2):
        acc2 = jnp.minimum(acc2, acc[:, t * _NU2:(t + 1) * _NU2])
    a2_ref[0] = acc2

    row0 = b * n + pl.program_id(1) * _BR
    ct_ref[0] = lax.broadcasted_iota(jnp.int32, (_BR, k), 0) + row0


def _topk_sc_body(dist_hbm, accm_hbm, acc2m_hbm, nn_hbm, buf, accb, acc2b,
                  outv, semd, sema, sem2, *, n, nrows, k):
    nc = 2
    wid = lax.axis_index("s") * nc + lax.axis_index("c")
    rpw = nrows // 32
    base = wid * rpw
    lanes = jnp.arange(_L, dtype=jnp.int32)
    tcap = jnp.minimum(lanes, _T - 1)
    t2cap = jnp.minimum(lanes, _T2 - 1)
    last = base + rpw - 1

    def fmin_s(v):
        return -(plsc.cummax(-v)[_L - 1])

    imin_s = fmin_s

    def start(half, row):
        pltpu.make_async_copy(dist_hbm.at[row], buf.at[pl.ds(half * n, n)],
                              semd).start()
        pltpu.make_async_copy(accm_hbm.at[row],
                              accb.at[pl.ds(half * _NU, _NU)], sema).start()
        pltpu.make_async_copy(acc2m_hbm.at[row],
                              acc2b.at[pl.ds(half * _NU2, _NU2)], sem2).start()

    def wait():
        pltpu.make_async_copy(dist_hbm.at[base], buf.at[pl.ds(0, n)],
                              semd).wait()
        pltpu.make_async_copy(accm_hbm.at[base], accb.at[pl.ds(0, _NU)],
                              sema).wait()
        pltpu.make_async_copy(acc2m_hbm.at[base], acc2b.at[pl.ds(0, _NU2)],
                              sem2).wait()

    def process(bb, ab, a2b, row):
        idxs = jnp.zeros((_L,), jnp.int32)
        for t in range(k + 1):
            cur = acc2b[pl.ds(a2b, _L)]
            jsel = jnp.zeros((_L,), jnp.int32)
            for a in range(1, _NU2 // _L):
                v = acc2b[pl.ds(a2b + a * _L, _L)]
                lt = v < cur
                cur = jnp.where(lt, v, cur)
                jsel = jnp.where(lt, a, jsel)
            m = fmin_s(cur)
            meq = cur == m
            jstar = imin_s(jnp.where(meq, jsel, _NU2 // _L))
            lstar = imin_s(jnp.where(meq & (jsel == jstar), lanes, _L))
            w = jstar * _L + lstar
            g1 = plsc.load_gather(accb, [ab + t2cap * _NU2 + w])
            t1 = imin_s(jnp.where(g1 == m, lanes, _L))
            u = t1 * _NU2 + w
            g0 = plsc.load_gather(buf, [bb + tcap * _NU + u])
            t0 = imin_s(jnp.where(g0 == m, lanes, _L))
            col = t0 * _NU + u
            vv = buf[pl.ds(bb + col - lstar, _L)]
            buf[pl.ds(bb + col - lstar, _L)] = jnp.where(
                lanes == lstar, jnp.inf, vv)
            r0 = buf[pl.ds(bb + u - lstar, _L)]
            for tt in range(1, _T):
                r0 = jnp.minimum(r0, buf[pl.ds(bb + tt * _NU + u - lstar, _L)])
            accb[pl.ds(ab + u - lstar, _L)] = r0
            r1 = accb[pl.ds(ab + w - lstar, _L)]
            for tt in range(1, _T2):
                r1 = jnp.minimum(
                    r1, accb[pl.ds(ab + tt * _NU2 + w - lstar, _L)])
            acc2b[pl.ds(a2b + w - lstar, _L)] = r1
            if t > 0:
                idxs = jnp.where(lanes == (t - 1), col, idxs)

        off = jnp.where(row >= n, n, 0)
        outv[...] = idxs + off
        pltpu.sync_copy(outv, nn_hbm.at[row])

    start(0, base)

    def outer(i, carry):
        p = jnp.bitwise_and(i, 1)
        row = base + i
        wait()
        start(1 - p, jnp.minimum(row + 1, last))
        process(p * n, p * _NU, p * _NU2, row)
        return carry
    lax.fori_loop(0, rpw, outer, 0)
    wait()


def kernel(x):
    B, C, H, W = x.shape
    n = H * W
    rows = B * n
    xc = x.reshape(B, C, n)
    xt = jnp.transpose(xc, (0, 2, 1))

    dist, accm, acc2m, ct = pl.pallas_call(
        functools.partial(_dist_body, n=n, k=_K),
        grid=(B, n // _BR),
        in_specs=[
            pl.BlockSpec((1, _BR, C), lambda b, r: (b, r, 0)),
            pl.BlockSpec((1, C, n), lambda b, r: (b, 0, 0)),
        ],
        out_specs=[
            pl.BlockSpec((1, _BR, n), lambda b, r: (b, r, 0)),
            pl.BlockSpec((1, _BR, _NU), lambda b, r: (b, r, 0)),
            pl.BlockSpec((1, _BR, _NU2), lambda b, r: (b, r, 0)),
            pl.BlockSpec((1, _BR, _K), lambda b, r: (b, r, 0)),
        ],
        out_shape=[
            jax.ShapeDtypeStruct((B, n, n), jnp.float32),
            jax.ShapeDtypeStruct((B, n, _NU), jnp.float32),
            jax.ShapeDtypeStruct((B, n, _NU2), jnp.float32),
            jax.ShapeDtypeStruct((B, n, _K), jnp.int32),
        ],
    )(xt, xc)

    mesh = plsc.VectorSubcoreMesh(core_axis_name="c", subcore_axis_name="s")
    nn = pl.kernel(
        functools.partial(_topk_sc_body, n=n, nrows=rows, k=_K),
        out_type=jax.ShapeDtypeStruct((rows, _K), jnp.int32),
        mesh=mesh,
        compiler_params=pltpu.CompilerParams(needs_layout_passes=False),
        scratch_types=[
            pltpu.VMEM((2 * n,), jnp.float32),
            pltpu.VMEM((2 * _NU,), jnp.float32),
            pltpu.VMEM((2 * _NU2,), jnp.float32),
            pltpu.VMEM((_L,), jnp.int32),
            pltpu.SemaphoreType.DMA,
            pltpu.SemaphoreType.DMA,
            pltpu.SemaphoreType.DMA,
        ],
    )(dist.reshape(rows, n), accm.reshape(rows, _NU),
      acc2m.reshape(rows, _NU2))
    return jnp.stack((nn.reshape(-1), ct.reshape(-1)), axis=0)


# speedup vs baseline: 43.9760x; 1.5360x over previous
import functools

import jax
import jax.numpy as jnp
from jax import lax
from jax.experimental import pallas as pl
from jax.experimental.pallas import tpu as pltpu
from jax.experimental.pallas import tpu_sc as plsc

_K = 16
_BR = 256
_L = 16
_T = 8
_NU = 1152
_T2 = 9
_NU2 = 128


def _dist_body(q_ref, k_ref, d_ref, a_ref, a2_ref, ct_ref, *, n, k):
    b = pl.program_id(0)
    q = q_ref[0]
    kt = k_ref[0]
    c_dim = q.shape[1]

    qs = q[:, 0:1] * q[:, 0:1]
    for c in range(1, c_dim):
        qs = qs + q[:, c:c + 1] * q[:, c:c + 1]
    qn = q / jnp.maximum(jnp.sqrt(qs), 1e-12)

    ks = kt[0:1] * kt[0:1]
    for c in range(1, c_dim):
        ks = ks + kt[c:c + 1] * kt[c:c + 1]
    knt = kt / jnp.maximum(jnp.sqrt(ks), 1e-12)

    dots = lax.dot_general(qn, knt, (((1,), (0,)), ((), ())),
                           preferred_element_type=jnp.float32)

    sq_q = qn[:, 0:1] * qn[:, 0:1]
    for c in range(1, c_dim):
        sq_q = sq_q + qn[:, c:c + 1] * qn[:, c:c + 1]
    sq_k = knt[0:1] * knt[0:1]
    for c in range(1, c_dim):
        sq_k = sq_k + knt[c:c + 1] * knt[c:c + 1]

    nd = -((sq_q + (-2.0 * dots)) + sq_k)
    d_ref[0] = nd

    acc = nd[:, 0:_NU]
    for t in range(1, _T):
        acc = jnp.maximum(acc, nd[:, t * _NU:(t + 1) * _NU])
    a_ref[0] = acc

    acc2 = acc[:, 0:_NU2]
    for t in range(1, _T2):
        acc2 = jnp.maximum(acc2, acc[:, t * _NU2:(t + 1) * _NU2])
    a2_ref[0] = acc2

    row0 = b * n + pl.program_id(1) * _BR
    ct_ref[0] = lax.broadcasted_iota(jnp.int32, (_BR, k), 0) + row0


def _topk_sc_body(dist_hbm, accm_hbm, acc2m_hbm, nn_hbm, buf, accb, acc2b,
                  outv, semd, sema, sem2, *, n, nrows, k):
    nc = 2
    wid = lax.axis_index("s") * nc + lax.axis_index("c")
    rpw = nrows // 32
    base = wid * rpw
    lanes = jnp.arange(_L, dtype=jnp.int32)
    tcap = jnp.minimum(lanes, _T - 1)
    t2cap = jnp.minimum(lanes, _T2 - 1)
    last = base + rpw - 1
    ninf = jnp.float32(-jnp.inf)

    def start(half, row):
        pltpu.make_async_copy(dist_hbm.at[row], buf.at[pl.ds(half * n, n)],
                              semd).start()
        pltpu.make_async_copy(accm_hbm.at[row],
                              accb.at[pl.ds(half * _NU, _NU)], sema).start()
        pltpu.make_async_copy(acc2m_hbm.at[row],
                              acc2b.at[pl.ds(half * _NU2, _NU2)], sem2).start()

    def wait():
        pltpu.make_async_copy(dist_hbm.at[base], buf.at[pl.ds(0, n)],
                              semd).wait()
        pltpu.make_async_copy(accm_hbm.at[base], accb.at[pl.ds(0, _NU)],
                              sema).wait()
        pltpu.make_async_copy(acc2m_hbm.at[base], acc2b.at[pl.ds(0, _NU2)],
                              sem2).wait()

    def process(h, row):
        bb, ab, a2b = h * n, h * _NU, h * _NU2

        def round_t(t, carry):
            (idxs,) = carry
            cur = acc2b[pl.ds(a2b, _L)]
            jsel = jnp.zeros((_L,), jnp.int32)
            for a in range(1, _NU2 // _L):
                v = acc2b[pl.ds(a2b + a * _L, _L)]
                gt = v > cur
                cur = jnp.where(gt, v, cur)
                jsel = jnp.where(gt, a, jsel)
            sk, sv = plsc.sort_key_val(cur, jsel * _L + lanes, descending=True)
            m = sk[0]
            w = sv[0]
            lstar = jnp.bitwise_and(w, _L - 1)
            g1 = plsc.load_gather(accb, [ab + t2cap * _NU2 + w])
            _, s1 = plsc.sort_key_val(g1, lanes, descending=True)
            u = s1[0] * _NU2 + w
            g0 = plsc.load_gather(buf, [bb + tcap * _NU + u])
            _, s0 = plsc.sort_key_val(g0, lanes, descending=True)
            col = s0[0] * _NU + u
            vv = buf[pl.ds(bb + col - lstar, _L)]
            buf[pl.ds(bb + col - lstar, _L)] = jnp.where(
                lanes == lstar, ninf, vv)
            r0 = buf[pl.ds(bb + u - lstar, _L)]
            for tt in range(1, _T):
                r0 = jnp.maximum(r0, buf[pl.ds(bb + tt * _NU + u - lstar, _L)])
            accb[pl.ds(ab + u - lstar, _L)] = r0
            r1 = accb[pl.ds(ab + w - lstar, _L)]
            for tt in range(1, _T2):
                r1 = jnp.maximum(
                    r1, accb[pl.ds(ab + tt * _NU2 + w - lstar, _L)])
            acc2b[pl.ds(a2b + w - lstar, _L)] = r1
            idxs = jnp.where(lanes == (t - 1), col, idxs)
            return (idxs,)

        (idxs,) = lax.fori_loop(0, k + 1, round_t,
                                (jnp.zeros((_L,), jnp.int32),))

        off = jnp.where(row >= n, n, 0)
        outv[...] = idxs + off
        pltpu.sync_copy(outv, nn_hbm.at[row])

    start(0, base)

    def outer(i, carry):
        p = jnp.bitwise_and(i, 1)
        row = base + i
        wait()
        start(1 - p, jnp.minimum(row + 1, last))
        process(p, row)
        return carry
    lax.fori_loop(0, rpw, outer, 0)
    wait()


def kernel(x):
    B, C, H, W = x.shape
    n = H * W
    rows = B * n
    xc = x.reshape(B, C, n)
    xt = jnp.transpose(xc, (0, 2, 1))

    dist, accm, acc2m, ct = pl.pallas_call(
        functools.partial(_dist_body, n=n, k=_K),
        grid=(B, n // _BR),
        in_specs=[
            pl.BlockSpec((1, _BR, C), lambda b, r: (b, r, 0)),
            pl.BlockSpec((1, C, n), lambda b, r: (b, 0, 0)),
        ],
        out_specs=[
            pl.BlockSpec((1, _BR, n), lambda b, r: (b, r, 0)),
            pl.BlockSpec((1, _BR, _NU), lambda b, r: (b, r, 0)),
            pl.BlockSpec((1, _BR, _NU2), lambda b, r: (b, r, 0)),
            pl.BlockSpec((1, _BR, _K), lambda b, r: (b, r, 0)),
        ],
        out_shape=[
            jax.ShapeDtypeStruct((B, n, n), jnp.float32),
            jax.ShapeDtypeStruct((B, n, _NU), jnp.float32),
            jax.ShapeDtypeStruct((B, n, _NU2), jnp.float32),
            jax.ShapeDtypeStruct((B, n, _K), jnp.int32),
        ],
    )(xt, xc)

    mesh = plsc.VectorSubcoreMesh(core_axis_name="c", subcore_axis_name="s")
    nn = pl.kernel(
        functools.partial(_topk_sc_body, n=n, nrows=rows, k=_K),
        out_type=jax.ShapeDtypeStruct((rows, _K), jnp.int32),
        mesh=mesh,
        compiler_params=pltpu.CompilerParams(needs_layout_passes=False),
        scratch_types=[
            pltpu.VMEM((2 * n,), jnp.float32),
            pltpu.VMEM((2 * _NU,), jnp.float32),
            pltpu.VMEM((2 * _NU2,), jnp.float32),
            pltpu.VMEM((_L,), jnp.int32),
            pltpu.SemaphoreType.DMA,
            pltpu.SemaphoreType.DMA,
            pltpu.SemaphoreType.DMA,
        ],
    )(dist.reshape(rows, n), accm.reshape(rows, _NU),
      acc2m.reshape(rows, _NU2))
    return jnp.stack((nn.reshape(-1), ct.reshape(-1)), axis=0)


# speedup vs baseline: 44.3911x; 1.0094x over previous
import functools

import jax
import jax.numpy as jnp
from jax import lax
from jax.experimental import pallas as pl
from jax.experimental.pallas import tpu as pltpu
from jax.experimental.pallas import tpu_sc as plsc

_K = 16
_BR = 256
_L = 16
_T = 8
_NU = 1152
_T2 = 9
_NU2 = 128


def _dist_body(q_ref, k_ref, d_ref, a_ref, a2_ref, ct_ref, *, n, k):
    b = pl.program_id(0)
    q = q_ref[0]
    kt = k_ref[0]
    c_dim = q.shape[1]

    qs = q[:, 0:1] * q[:, 0:1]
    for c in range(1, c_dim):
        qs = qs + q[:, c:c + 1] * q[:, c:c + 1]
    qn = q / jnp.maximum(jnp.sqrt(qs), 1e-12)

    ks = kt[0:1] * kt[0:1]
    for c in range(1, c_dim):
        ks = ks + kt[c:c + 1] * kt[c:c + 1]
    knt = kt / jnp.maximum(jnp.sqrt(ks), 1e-12)

    dots = lax.dot_general(qn, knt, (((1,), (0,)), ((), ())),
                           preferred_element_type=jnp.float32)

    sq_q = qn[:, 0:1] * qn[:, 0:1]
    for c in range(1, c_dim):
        sq_q = sq_q + qn[:, c:c + 1] * qn[:, c:c + 1]
    sq_k = knt[0:1] * knt[0:1]
    for c in range(1, c_dim):
        sq_k = sq_k + knt[c:c + 1] * knt[c:c + 1]

    nd = -((sq_q + (-2.0 * dots)) + sq_k)
    d_ref[0] = nd

    acc = nd[:, 0:_NU]
    for t in range(1, _T):
        acc = jnp.maximum(acc, nd[:, t * _NU:(t + 1) * _NU])
    a_ref[0] = acc

    acc2 = acc[:, 0:_NU2]
    for t in range(1, _T2):
        acc2 = jnp.maximum(acc2, acc[:, t * _NU2:(t + 1) * _NU2])
    a2_ref[0] = acc2

    row0 = b * n + pl.program_id(1) * _BR
    ct_ref[0] = lax.broadcasted_iota(jnp.int32, (_BR, k), 0) + row0


def _topk_sc_body(dist_hbm, accm_hbm, acc2m_hbm, nn_hbm, buf, accb, acc2b,
                  outv, bufB, accbB, acc2bB, outvB, semd, sema, sem2,
                  semdB, semaB, sem2B, *, n, nrows, k):
    nc = 2
    wid = lax.axis_index("s") * nc + lax.axis_index("c")
    rpw = nrows // 32
    base = wid * rpw
    lanes = jnp.arange(_L, dtype=jnp.int32)
    tcap = jnp.minimum(lanes, _T - 1)
    t2cap = jnp.minimum(lanes, _T2 - 1)
    last = base + rpw - 1
    ninf = jnp.float32(-jnp.inf)

    def start(b0, b1, b2, s0, s1, s2, half, row):
        pltpu.make_async_copy(dist_hbm.at[row], b0.at[pl.ds(half * n, n)],
                              s0).start()
        pltpu.make_async_copy(accm_hbm.at[row],
                              b1.at[pl.ds(half * _NU, _NU)], s1).start()
        pltpu.make_async_copy(acc2m_hbm.at[row],
                              b2.at[pl.ds(half * _NU2, _NU2)], s2).start()

    def wait(b0, b1, b2, s0, s1, s2):
        pltpu.make_async_copy(dist_hbm.at[base], b0.at[pl.ds(0, n)],
                              s0).wait()
        pltpu.make_async_copy(accm_hbm.at[base], b1.at[pl.ds(0, _NU)],
                              s1).wait()
        pltpu.make_async_copy(acc2m_hbm.at[base], b2.at[pl.ds(0, _NU2)],
                              s2).wait()

    def process(h, rowA, rowB):
        bb, ab, a2b = h * n, h * _NU, h * _NU2

        def one(buf, accb, acc2b, t, idxs):
            cur = acc2b[pl.ds(a2b, _L)]
            jsel = jnp.zeros((_L,), jnp.int32)
            for a in range(1, _NU2 // _L):
                v = acc2b[pl.ds(a2b + a * _L, _L)]
                gt = v > cur
                cur = jnp.where(gt, v, cur)
                jsel = jnp.where(gt, a, jsel)
            sk, sv = plsc.sort_key_val(cur, jsel * _L + lanes, descending=True)
            m = sk[0]
            w = sv[0]
            lstar = jnp.bitwise_and(w, _L - 1)
            g1 = plsc.load_gather(accb, [ab + t2cap * _NU2 + w])
            _, s1 = plsc.sort_key_val(g1, lanes, descending=True)
            u = s1[0] * _NU2 + w
            g0 = plsc.load_gather(buf, [bb + tcap * _NU + u])
            _, s0 = plsc.sort_key_val(g0, lanes, descending=True)
            col = s0[0] * _NU + u
            vv = buf[pl.ds(bb + col - lstar, _L)]
            buf[pl.ds(bb + col - lstar, _L)] = jnp.where(
                lanes == lstar, ninf, vv)
            r0 = buf[pl.ds(bb + u - lstar, _L)]
            for tt in range(1, _T):
                r0 = jnp.maximum(r0, buf[pl.ds(bb + tt * _NU + u - lstar, _L)])
            accb[pl.ds(ab + u - lstar, _L)] = r0
            r1 = accb[pl.ds(ab + w - lstar, _L)]
            for tt in range(1, _T2):
                r1 = jnp.maximum(
                    r1, accb[pl.ds(ab + tt * _NU2 + w - lstar, _L)])
            acc2b[pl.ds(a2b + w - lstar, _L)] = r1
            return jnp.where(lanes == (t - 1), col, idxs)

        def round_t(t, carry):
            idxsA, idxsB = carry
            idxsA = one(buf, accb, acc2b, t, idxsA)
            idxsB = one(bufB, accbB, acc2bB, t, idxsB)
            return idxsA, idxsB

        idxsA, idxsB = lax.fori_loop(
            0, k + 1, round_t,
            (jnp.zeros((_L,), jnp.int32), jnp.zeros((_L,), jnp.int32)))

        offA = jnp.where(rowA >= n, n, 0)
        offB = jnp.where(rowB >= n, n, 0)
        outv[...] = idxsA + offA
        outvB[...] = idxsB + offB
        pltpu.sync_copy(outv, nn_hbm.at[rowA])
        pltpu.sync_copy(outvB, nn_hbm.at[rowB])

    start(buf, accb, acc2b, semd, sema, sem2, 0, base)
    start(bufB, accbB, acc2bB, semdB, semaB, sem2B, 0, base + 1)

    def outer(i, carry):
        p = jnp.bitwise_and(i, 1)
        rowA = base + 2 * i
        rowB = rowA + 1
        wait(buf, accb, acc2b, semd, sema, sem2)
        wait(bufB, accbB, acc2bB, semdB, semaB, sem2B)
        start(buf, accb, acc2b, semd, sema, sem2, 1 - p,
              jnp.minimum(rowA + 2, last))
        start(bufB, accbB, acc2bB, semdB, semaB, sem2B, 1 - p,
              jnp.minimum(rowB + 2, last))
        process(p, rowA, rowB)
        return carry
    lax.fori_loop(0, rpw // 2, outer, 0)
    wait(buf, accb, acc2b, semd, sema, sem2)
    wait(bufB, accbB, acc2bB, semdB, semaB, sem2B)


def kernel(x):
    B, C, H, W = x.shape
    n = H * W
    rows = B * n
    xc = x.reshape(B, C, n)
    xt = jnp.transpose(xc, (0, 2, 1))

    dist, accm, acc2m, ct = pl.pallas_call(
        functools.partial(_dist_body, n=n, k=_K),
        grid=(B, n // _BR),
        in_specs=[
            pl.BlockSpec((1, _BR, C), lambda b, r: (b, r, 0)),
            pl.BlockSpec((1, C, n), lambda b, r: (b, 0, 0)),
        ],
        out_specs=[
            pl.BlockSpec((1, _BR, n), lambda b, r: (b, r, 0)),
            pl.BlockSpec((1, _BR, _NU), lambda b, r: (b, r, 0)),
            pl.BlockSpec((1, _BR, _NU2), lambda b, r: (b, r, 0)),
            pl.BlockSpec((1, _BR, _K), lambda b, r: (b, r, 0)),
        ],
        out_shape=[
            jax.ShapeDtypeStruct((B, n, n), jnp.float32),
            jax.ShapeDtypeStruct((B, n, _NU), jnp.float32),
            jax.ShapeDtypeStruct((B, n, _NU2), jnp.float32),
            jax.ShapeDtypeStruct((B, n, _K), jnp.int32),
        ],
    )(xt, xc)

    mesh = plsc.VectorSubcoreMesh(core_axis_name="c", subcore_axis_name="s")
    nn = pl.kernel(
        functools.partial(_topk_sc_body, n=n, nrows=rows, k=_K),
        out_type=jax.ShapeDtypeStruct((rows, _K), jnp.int32),
        mesh=mesh,
        compiler_params=pltpu.CompilerParams(needs_layout_passes=False),
        scratch_types=[
            pltpu.VMEM((2 * n,), jnp.float32),
            pltpu.VMEM((2 * _NU,), jnp.float32),
            pltpu.VMEM((2 * _NU2,), jnp.float32),
            pltpu.VMEM((_L,), jnp.int32),
            pltpu.VMEM((2 * n,), jnp.float32),
            pltpu.VMEM((2 * _NU,), jnp.float32),
            pltpu.VMEM((2 * _NU2,), jnp.float32),
            pltpu.VMEM((_L,), jnp.int32),
            pltpu.SemaphoreType.DMA,
            pltpu.SemaphoreType.DMA,
            pltpu.SemaphoreType.DMA,
            pltpu.SemaphoreType.DMA,
            pltpu.SemaphoreType.DMA,
            pltpu.SemaphoreType.DMA,
        ],
    )(dist.reshape(rows, n), accm.reshape(rows, _NU),
      acc2m.reshape(rows, _NU2))
    return jnp.stack((nn.reshape(-1), ct.reshape(-1)), axis=0)
